# Initial kernel scaffold; baseline (speedup 1.0000x reference)
import jax
import jax.numpy as jnp
from jax.experimental import pallas as pl


def kernel(hidden_states, weight, bias):
    raise NotImplementedError("write your pallas kernel here")


# speedup vs baseline: 1.0190x; 1.0190x over previous
import functools

import jax
import jax.numpy as jnp
from jax import lax
from jax.experimental import pallas as pl
from jax.experimental.pallas import tpu as pltpu
from jax.experimental.pallas import tpu_sc as plsc

TOP_K = 8
L = 16
NC, NS = 2, 16
NW = NC * NS


def _matmul_body(x_ref, w_ref, b_ref, out_ref, outT_ref):
    acc = jnp.dot(x_ref[...], w_ref[...], preferred_element_type=jnp.float32)
    acc = acc + b_ref[...]
    out_ref[...] = acc
    outT_ref[...] = acc.T


def _router_logits(x, w_t, bias2d, block_tokens):
    tokens, hidden = x.shape
    experts = w_t.shape[1]
    return pl.pallas_call(
        _matmul_body,
        grid=(tokens // block_tokens,),
        in_specs=[
            pl.BlockSpec((block_tokens, hidden), lambda i: (i, 0)),
            pl.BlockSpec((hidden, experts), lambda i: (0, 0)),
            pl.BlockSpec((1, experts), lambda i: (0, 0)),
        ],
        out_specs=[
            pl.BlockSpec((block_tokens, experts), lambda i: (i, 0)),
            pl.BlockSpec((experts, block_tokens), lambda i: (0, i)),
        ],
        out_shape=[
            jax.ShapeDtypeStruct((tokens, experts), jnp.float32),
            jax.ShapeDtypeStruct((experts, tokens), jnp.float32),
        ],
        compiler_params=pltpu.CompilerParams(
            dimension_semantics=("arbitrary",),
        ),
    )(x, w_t, bias2d)


def _make_topk_sc(tokens, experts):
    tw = tokens // NW
    groups = tw // L

    def body(lt_ref, scores_ref, idx_ref, tile, obuf_s, obuf_i, sem):
        neg_inf = jnp.full((L,), -jnp.inf, dtype=jnp.float32)
        c = lax.axis_index("c")
        s = lax.axis_index("s")
        wid = s * NC + c
        t0 = wid * tw
        copies = [
            pltpu.async_copy(lt_ref.at[j, pl.ds(t0, tw)],
                             tile.at[pl.ds(j * tw, tw)], sem)
            for j in range(experts)
        ]
        for cp in copies:
            cp.wait()
        lanes = lax.iota(jnp.int32, L)

        def group_body(g, carry):
            col = g * L
            toks = col + lanes
            toks_k = toks * TOP_K
            ms, ixs = [], []
            for _ in range(TOP_K):
                accs = []
                for blk in range(8):
                    j0 = blk * 8
                    m = tile[pl.ds(j0 * tw + col, L)]
                    ix = jnp.full((L,), j0, dtype=jnp.int32)
                    for j in range(j0 + 1, j0 + 8):
                        v = tile[pl.ds(j * tw + col, L)]
                        p = v > m
                        m = jnp.where(p, v, m)
                        ix = jnp.where(p, jnp.full((L,), j, dtype=jnp.int32), ix)
                    accs.append((m, ix))
                while len(accs) > 1:
                    nxt = []
                    for a, b in zip(accs[0::2], accs[1::2]):
                        p = b[0] > a[0]
                        nxt.append((jnp.where(p, b[0], a[0]),
                                    jnp.where(p, b[1], a[1])))
                    accs = nxt
                m, ix = accs[0]
                ms.append(m)
                ixs.append(ix)
                plsc.store_scatter(tile, [ix * tw + toks], neg_inf)
            es = [jnp.exp(mm - ms[0]) for mm in ms]
            tot = es[0]
            for e in es[1:]:
                tot = tot + e
            inv = 1.0 / tot
            for r in range(TOP_K):
                plsc.store_scatter(obuf_s, [toks_k + r], es[r] * inv)
                plsc.store_scatter(obuf_i, [toks_k + r], ixs[r])
            return carry

        lax.fori_loop(0, groups, group_body, 0)
        pltpu.sync_copy(obuf_s, scores_ref.at[pl.ds(t0 * TOP_K, tw * TOP_K)])
        pltpu.sync_copy(obuf_i, idx_ref.at[pl.ds(t0 * TOP_K, tw * TOP_K)])

    return pl.kernel(
        body,
        out_type=[
            jax.ShapeDtypeStruct((tokens * TOP_K,), jnp.float32),
            jax.ShapeDtypeStruct((tokens * TOP_K,), jnp.int32),
        ],
        mesh=plsc.VectorSubcoreMesh(core_axis_name="c", subcore_axis_name="s"),
        compiler_params=pltpu.CompilerParams(use_tc_tiling_on_sc=False,
                                             needs_layout_passes=False),
        scratch_types=[
            pltpu.VMEM((experts * tw,), jnp.float32),
            pltpu.VMEM((tw * TOP_K,), jnp.float32),
            pltpu.VMEM((tw * TOP_K,), jnp.int32),
            pltpu.SemaphoreType.DMA,
        ],
    )


@jax.jit
def kernel(hidden_states, weight, bias):
    tokens, _ = hidden_states.shape
    experts = weight.shape[0]
    w_t = weight.T
    bias2d = bias.reshape(1, experts)
    logits, logits_t = _router_logits(hidden_states, w_t, bias2d,
                                      block_tokens=512)
    scores_flat, idx_flat = _make_topk_sc(tokens, experts)(logits_t)
    return (logits,
            scores_flat.reshape(tokens, TOP_K),
            idx_flat.reshape(tokens, TOP_K))
